# baseline (speedup 1.0000x reference)
import functools

import jax
import jax.numpy as jnp
from jax import lax
from jax.experimental import pallas as pl
from jax.experimental.pallas import tpu as pltpu
from jax.experimental.pallas import tpu_sc as plsc

N = 50000
E = 800000
DIN = 128
H = 64
G = 512
HH = H // 2

NSC = 2
NT = 16
RPT = N // NT

EP = 16 * 1024 * 49
EPT = EP // NT
ROWS_PT = EPT // 128
CJ = 2
CHUNKS = ROWS_PT // CJ

SP_ROWS = N + 8

CR = 128
GPT = G // NT

_mesh = plsc.VectorSubcoreMesh(core_axis_name="c", subcore_axis_name="s")


def _agg_body(g2, src2, dstp, out, srcvA, dstvA, srcvB, dstvB, rows, rowsB,
              spo, sem):
    c = lax.axis_index("c")
    s = lax.axis_index("s")
    base = c * N + s * RPT
    pltpu.sync_copy(g2.at[pl.ds(base, RPT)], spo.at[pl.ds(s * RPT, RPT)])
    plsc.subcore_barrier()

    src_row0 = c * (EP // 128) + s * ROWS_PT
    dst_row0 = s * ROWS_PT

    pltpu.sync_copy(src2.at[pl.ds(src_row0, CJ)], srcvA)
    pltpu.sync_copy(dstp.at[pl.ds(dst_row0, CJ)], dstvA)

    def pair(k, carry):
        cpsA = [pltpu.async_copy(g2.at[srcvA.at[j]], rows.at[j], sem)
                for j in range(CJ)]
        pltpu.sync_copy(src2.at[pl.ds(src_row0 + (2 * k + 1) * CJ, CJ)], srcvB)
        pltpu.sync_copy(dstp.at[pl.ds(dst_row0 + (2 * k + 1) * CJ, CJ)], dstvB)
        cpsB = [pltpu.async_copy(g2.at[srcvB.at[j]], rowsB.at[j], sem)
                for j in range(CJ)]
        for j in range(CJ):
            cpsA[j].wait()
            pltpu.sync_copy(rows.at[j], spo.at[dstvA.at[j]], add=True)
        nxt = jnp.minimum((2 * k + 2) * CJ, (CHUNKS - 2) * CJ)
        pltpu.sync_copy(src2.at[pl.ds(src_row0 + nxt, CJ)], srcvA)
        pltpu.sync_copy(dstp.at[pl.ds(dst_row0 + nxt, CJ)], dstvA)
        for j in range(CJ):
            cpsB[j].wait()
            pltpu.sync_copy(rowsB.at[j], spo.at[dstvB.at[j]], add=True)
        return carry

    lax.fori_loop(0, CHUNKS // 2, pair, 0)
    plsc.subcore_barrier()
    pltpu.sync_copy(spo.at[pl.ds(s * RPT, RPT)], out.at[pl.ds(base, RPT)])


_agg = functools.partial(
    pl.kernel,
    out_type=jax.ShapeDtypeStruct((2 * N, HH), jnp.float32),
    mesh=_mesh,
    compiler_params=pltpu.CompilerParams(use_tc_tiling_on_sc=False),
    scratch_types=[
        pltpu.VMEM((CJ, 128), jnp.int32),
        pltpu.VMEM((CJ, 128), jnp.int32),
        pltpu.VMEM((CJ, 128), jnp.int32),
        pltpu.VMEM((CJ, 128), jnp.int32),
        pltpu.VMEM((CJ, 128, HH), jnp.float32),
        pltpu.VMEM((CJ, 128, HH), jnp.float32),
        pltpu.VMEM_SHARED((SP_ROWS, HH), jnp.float32),
        pltpu.SemaphoreType.DMA,
    ],
)(_agg_body)


DHH = 16
DROWS = EP // 128 // 2
DRPT = DROWS // NT
DCJ = 4
DCHUNKS = DRPT // DCJ


def _deg_body(dstp, out, dstv, ones_rows, spo):
    c = lax.axis_index("c")
    s = lax.axis_index("s")
    one16 = jnp.ones((DHH,), jnp.float32)

    def oi(i, carry):
        ones_rows[i] = one16
        return carry

    lax.fori_loop(0, 128, oi, 0)

    def zi(i, carry):
        pltpu.sync_copy(ones_rows.at[pl.ds(0, 125)],
                        spo.at[pl.ds(s * RPT + i * 125, 125)])
        return carry

    lax.fori_loop(0, 25, zi, 0)
    plsc.subcore_barrier()

    row0 = c * DROWS + s * DRPT

    def chunk(i, carry):
        pltpu.sync_copy(dstp.at[pl.ds(row0 + i * DCJ, DCJ)], dstv)
        for j in range(DCJ):
            pltpu.sync_copy(ones_rows.at[pl.ds(0, 128)], spo.at[dstv.at[j]],
                            add=True)
        return carry

    lax.fori_loop(0, DCHUNKS, chunk, 0)
    plsc.subcore_barrier()
    pltpu.sync_copy(spo.at[pl.ds(s * RPT, RPT)], out.at[pl.ds(c * N + s * RPT, RPT)])


_deg = functools.partial(
    pl.kernel,
    out_type=jax.ShapeDtypeStruct((2 * N, DHH), jnp.float32),
    mesh=_mesh,
    compiler_params=pltpu.CompilerParams(use_tc_tiling_on_sc=False),
    scratch_types=[
        pltpu.VMEM((DCJ, 128), jnp.int32),
        pltpu.VMEM((128, DHH), jnp.float32),
        pltpu.VMEM_SHARED((SP_ROWS, DHH), jnp.float32),
    ],
)(_deg_body)


def _pool_body(hid2, bounds, pooled4, bbuf, rbuf, resmax, resmean):
    c = lax.axis_index("c")
    s = lax.axis_index("s")
    pltpu.sync_copy(bounds.at[pl.ds(s * GPT, 48)], bbuf)

    lane = lax.broadcasted_iota(jnp.int32, (16,), 0)

    def extract(k):
        blk = bbuf[pl.ds((k // 16) * 16, 16)]
        sel = lane == lax.broadcast(k % 16, (16,))
        return jnp.sum(jnp.where(sel, blk, 0))

    neg = jnp.full((16,), -jnp.inf, dtype=jnp.float32)
    zero = jnp.zeros((16,), jnp.float32)

    def graph(k, carry):
        start = extract(k)
        end = extract(k + 1)
        nch = (end - start + CR - 1) // CR

        def chunk(t, acc):
            ofs = jnp.minimum(start + t * CR, N - CR)
            lo = start + t * CR
            pltpu.sync_copy(hid2.at[pl.ds(c * N + ofs, CR)], rbuf)

            def row(r, acc2):
                m0, m1, s0, s1 = acc2
                rid = ofs + r
                valid = jnp.logical_and(rid >= lo, rid < end)
                vb = lax.broadcast(valid, (16,))
                v0 = rbuf[r, pl.ds(0, 16)]
                v1 = rbuf[r, pl.ds(16, 16)]
                m0 = jnp.maximum(m0, jnp.where(vb, v0, neg))
                m1 = jnp.maximum(m1, jnp.where(vb, v1, neg))
                s0 = s0 + jnp.where(vb, v0, zero)
                s1 = s1 + jnp.where(vb, v1, zero)
                return (m0, m1, s0, s1)

            return lax.fori_loop(0, CR, row, acc)

        m0, m1, s0, s1 = lax.fori_loop(0, nch, chunk, (neg, neg, zero, zero))
        resmax[k, pl.ds(0, 16)] = m0
        resmax[k, pl.ds(16, 16)] = m1
        resmean[k, pl.ds(0, 16)] = s0
        resmean[k, pl.ds(16, 16)] = s1
        return carry

    lax.fori_loop(0, GPT, graph, 0)
    out_base = c * G + s * GPT
    pltpu.sync_copy(resmax, pooled4.at[pl.ds(out_base, GPT)])
    pltpu.sync_copy(resmean, pooled4.at[pl.ds(2 * G + out_base, GPT)])


_pool = functools.partial(
    pl.kernel,
    out_type=jax.ShapeDtypeStruct((4 * G, HH), jnp.float32),
    mesh=_mesh,
    compiler_params=pltpu.CompilerParams(use_tc_tiling_on_sc=False,
                                         needs_layout_passes=False),
    scratch_types=[
        pltpu.VMEM((48,), jnp.int32),
        pltpu.VMEM((CR, HH), jnp.float32),
        pltpu.VMEM((GPT, HH), jnp.float32),
        pltpu.VMEM((GPT, HH), jnp.float32),
    ],
)(_pool_body)


R = 1000


def _tck0_body(x_ref, dega_ref, degb_ref, wa_ref, wb_ref, g_ref, dinv_ref):
    deg = dega_ref[...] + degb_ref[...] - 1.0
    dinv = lax.rsqrt(jnp.maximum(deg, 1.0))
    xb = x_ref[...]
    g_ref[0] = dinv * jnp.dot(xb, wa_ref[...], preferred_element_type=jnp.float32)
    g_ref[1] = dinv * jnp.dot(xb, wb_ref[...], preferred_element_type=jnp.float32)
    dinv_ref[...] = dinv


def _tck0(x, dega, degb, w0a, w0b):
    return pl.pallas_call(
        _tck0_body,
        grid=(N // R,),
        in_specs=[
            pl.BlockSpec((R, DIN), lambda i: (i, 0)),
            pl.BlockSpec((R, 1), lambda i: (i, 0)),
            pl.BlockSpec((R, 1), lambda i: (i, 0)),
            pl.BlockSpec((DIN, HH), lambda i: (0, 0)),
            pl.BlockSpec((DIN, HH), lambda i: (0, 0)),
        ],
        out_specs=[
            pl.BlockSpec((2, R, HH), lambda i: (0, i, 0)),
            pl.BlockSpec((R, 1), lambda i: (i, 0)),
        ],
        out_shape=[
            jax.ShapeDtypeStruct((2, N, HH), jnp.float32),
            jax.ShapeDtypeStruct((N, 1), jnp.float32),
        ],
    )(x, dega, degb, w0a, w0b)


def _tckmid_body(a_ref, dinv_ref, ba_ref, bb_ref, waa, wab, wba, wbb, g_ref):
    dinv = dinv_ref[...]
    xa = jnp.tanh(dinv * a_ref[0] + ba_ref[...])
    xb = jnp.tanh(dinv * a_ref[1] + bb_ref[...])
    ya = (jnp.dot(xa, waa[...], preferred_element_type=jnp.float32)
          + jnp.dot(xb, wba[...], preferred_element_type=jnp.float32))
    yb = (jnp.dot(xa, wab[...], preferred_element_type=jnp.float32)
          + jnp.dot(xb, wbb[...], preferred_element_type=jnp.float32))
    g_ref[0] = dinv * ya
    g_ref[1] = dinv * yb


def _tckmid(a3, dinv, b, W):
    return pl.pallas_call(
        _tckmid_body,
        grid=(N // R,),
        in_specs=[
            pl.BlockSpec((2, R, HH), lambda i: (0, i, 0)),
            pl.BlockSpec((R, 1), lambda i: (i, 0)),
            pl.BlockSpec((1, HH), lambda i: (0, 0)),
            pl.BlockSpec((1, HH), lambda i: (0, 0)),
            pl.BlockSpec((HH, HH), lambda i: (0, 0)),
            pl.BlockSpec((HH, HH), lambda i: (0, 0)),
            pl.BlockSpec((HH, HH), lambda i: (0, 0)),
            pl.BlockSpec((HH, HH), lambda i: (0, 0)),
        ],
        out_specs=pl.BlockSpec((2, R, HH), lambda i: (0, i, 0)),
        out_shape=jax.ShapeDtypeStruct((2, N, HH), jnp.float32),
    )(a3, dinv, b[:HH].reshape(1, HH), b[HH:].reshape(1, HH),
      W[:HH, :HH], W[:HH, HH:], W[HH:, :HH], W[HH:, HH:])


def _tckf_body(a_ref, dinv_ref, ba_ref, bb_ref, h_ref):
    dinv = dinv_ref[...]
    h_ref[0] = jnp.tanh(dinv * a_ref[0] + ba_ref[...])
    h_ref[1] = jnp.tanh(dinv * a_ref[1] + bb_ref[...])


def _tckf(a3, dinv, b):
    return pl.pallas_call(
        _tckf_body,
        grid=(N // R,),
        in_specs=[
            pl.BlockSpec((2, R, HH), lambda i: (0, i, 0)),
            pl.BlockSpec((R, 1), lambda i: (i, 0)),
            pl.BlockSpec((1, HH), lambda i: (0, 0)),
            pl.BlockSpec((1, HH), lambda i: (0, 0)),
        ],
        out_specs=pl.BlockSpec((2, R, HH), lambda i: (0, i, 0)),
        out_shape=jax.ShapeDtypeStruct((2, N, HH), jnp.float32),
    )(a3, dinv, b[:HH].reshape(1, HH), b[HH:].reshape(1, HH))


def _tckt_body(p_ref, r_ref, w_ref, b_ref, pooled_ref, out_ref):
    p4 = p_ref[...]
    r = r_ref[...]
    pooled = jnp.concatenate(
        [p4[0:G], p4[G:2 * G], p4[2 * G:3 * G] * r, p4[3 * G:] * r], axis=1)
    pooled_ref[...] = pooled
    out_ref[...] = (jnp.dot(pooled, w_ref[...],
                            preferred_element_type=jnp.float32) + b_ref[...])


def _tckt(pooled4, r, Wout, bout):
    return pl.pallas_call(
        _tckt_body,
        out_shape=[
            jax.ShapeDtypeStruct((G, 2 * H), jnp.float32),
            jax.ShapeDtypeStruct((G, 1), jnp.float32),
        ],
    )(pooled4, r, Wout, bout.reshape(1, 1))


def kernel(x, edge_index, batch_index, W0, b0, W1, b1, W2, b2, W3, b3, Wout, bout):
    src = edge_index[0].astype(jnp.int32)
    dst = edge_index[1].astype(jnp.int32)
    bi = batch_index.astype(jnp.int32)

    pad = EP - E
    src_p = jnp.concatenate([src, jnp.zeros((pad,), jnp.int32)])
    src2 = jnp.concatenate([src_p, src_p + N]).reshape(2 * EP // 128, 128)
    dstp = jnp.concatenate([dst, jnp.full((pad,), N, jnp.int32)]).reshape(EP // 128, 128)
    bounds = jnp.searchsorted(bi, jnp.arange(G + 1, dtype=jnp.int32)).astype(jnp.int32)
    bounds_p = jnp.concatenate([bounds, jnp.full((15,), N, jnp.int32)])

    degw = _deg(dstp)
    g0, dinv = _tck0(x, degw[:N, :1], degw[N:, :1], W0[:, :HH], W0[:, HH:])
    a = _agg(g0.reshape(2 * N, HH), src2, dstp)
    g1 = _tckmid(a.reshape(2, N, HH), dinv, b0, W1)
    a = _agg(g1.reshape(2 * N, HH), src2, dstp)
    g2 = _tckmid(a.reshape(2, N, HH), dinv, b1, W2)
    a = _agg(g2.reshape(2 * N, HH), src2, dstp)
    g3 = _tckmid(a.reshape(2, N, HH), dinv, b2, W3)
    a = _agg(g3.reshape(2 * N, HH), src2, dstp)
    hid = _tckf(a.reshape(2, N, HH), dinv, b3)

    pooled4 = _pool(hid.reshape(2 * N, HH), bounds_p)
    cnt = (bounds[1:] - bounds[:-1]).astype(jnp.float32)
    r = (1.0 / jnp.maximum(cnt, 1.0)).reshape(G, 1)
    pooled, out = _tckt(pooled4, r, Wout, bout)
    return (out, pooled)


# speedup vs baseline: 1.1507x; 1.1507x over previous
import functools

import jax
import jax.numpy as jnp
from jax import lax
from jax.experimental import pallas as pl
from jax.experimental.pallas import tpu as pltpu
from jax.experimental.pallas import tpu_sc as plsc

N = 50000
E = 800000
DIN = 128
H = 64
G = 512
HH = H // 2

NSC = 2
NT = 16
RPT = N // NT

EP = 16 * 1024 * 49
EPT = EP // NT
ROWS_PT = EPT // 128
CJ = 4
CHUNKS = ROWS_PT // CJ

SP_ROWS = N + 8

CR = 128
GPT = G // NT

_mesh = plsc.VectorSubcoreMesh(core_axis_name="c", subcore_axis_name="s")


def _agg_body(g2, src2, dstp, out, srcvA, dstvA, srcvB, dstvB, rows, spo, sem):
    c = lax.axis_index("c")
    s = lax.axis_index("s")
    base = c * N + s * RPT
    pltpu.sync_copy(g2.at[pl.ds(base, RPT)], spo.at[pl.ds(s * RPT, RPT)])
    plsc.subcore_barrier()

    src_row0 = c * (EP // 128) + s * ROWS_PT
    dst_row0 = s * ROWS_PT

    pltpu.sync_copy(src2.at[pl.ds(src_row0, CJ)], srcvA)
    pltpu.sync_copy(dstp.at[pl.ds(dst_row0, CJ)], dstvA)

    def pair(k, carry):
        cps = [pltpu.async_copy(g2.at[srcvA.at[j]], rows.at[j], sem)
               for j in range(CJ)]
        pltpu.sync_copy(src2.at[pl.ds(src_row0 + (2 * k + 1) * CJ, CJ)], srcvB)
        pltpu.sync_copy(dstp.at[pl.ds(dst_row0 + (2 * k + 1) * CJ, CJ)], dstvB)
        for j in range(CJ):
            cps[j].wait()
            pltpu.sync_copy(rows.at[j], spo.at[dstvA.at[j]], add=True)
        cps = [pltpu.async_copy(g2.at[srcvB.at[j]], rows.at[j], sem)
               for j in range(CJ)]
        nxt = jnp.minimum((2 * k + 2) * CJ, (CHUNKS - 2) * CJ)
        pltpu.sync_copy(src2.at[pl.ds(src_row0 + nxt, CJ)], srcvA)
        pltpu.sync_copy(dstp.at[pl.ds(dst_row0 + nxt, CJ)], dstvA)
        for j in range(CJ):
            cps[j].wait()
            pltpu.sync_copy(rows.at[j], spo.at[dstvB.at[j]], add=True)
        return carry

    lax.fori_loop(0, CHUNKS // 2, pair, 0)
    plsc.subcore_barrier()
    pltpu.sync_copy(spo.at[pl.ds(s * RPT, RPT)], out.at[pl.ds(base, RPT)])


_agg = functools.partial(
    pl.kernel,
    out_type=jax.ShapeDtypeStruct((2 * N, HH), jnp.float32),
    mesh=_mesh,
    compiler_params=pltpu.CompilerParams(use_tc_tiling_on_sc=False),
    scratch_types=[
        pltpu.VMEM((CJ, 128), jnp.int32),
        pltpu.VMEM((CJ, 128), jnp.int32),
        pltpu.VMEM((CJ, 128), jnp.int32),
        pltpu.VMEM((CJ, 128), jnp.int32),
        pltpu.VMEM((CJ, 128, HH), jnp.float32),
        pltpu.VMEM_SHARED((SP_ROWS, HH), jnp.float32),
        pltpu.SemaphoreType.DMA,
    ],
)(_agg_body)


DHH = 16
DROWS = EP // 128 // 2
DRPT = DROWS // NT
DCJ = 4
DCHUNKS = DRPT // DCJ


def _deg_body(dstp, out, dstv, ones_rows, spo):
    c = lax.axis_index("c")
    s = lax.axis_index("s")
    one16 = jnp.ones((DHH,), jnp.float32)

    def oi(i, carry):
        ones_rows[i] = one16
        return carry

    lax.fori_loop(0, 128, oi, 0)

    def zi(i, carry):
        pltpu.sync_copy(ones_rows.at[pl.ds(0, 125)],
                        spo.at[pl.ds(s * RPT + i * 125, 125)])
        return carry

    lax.fori_loop(0, 25, zi, 0)
    plsc.subcore_barrier()

    row0 = c * DROWS + s * DRPT

    def chunk(i, carry):
        pltpu.sync_copy(dstp.at[pl.ds(row0 + i * DCJ, DCJ)], dstv)
        for j in range(DCJ):
            pltpu.sync_copy(ones_rows.at[pl.ds(0, 128)], spo.at[dstv.at[j]],
                            add=True)
        return carry

    lax.fori_loop(0, DCHUNKS, chunk, 0)
    plsc.subcore_barrier()
    pltpu.sync_copy(spo.at[pl.ds(s * RPT, RPT)], out.at[pl.ds(c * N + s * RPT, RPT)])


_deg = functools.partial(
    pl.kernel,
    out_type=jax.ShapeDtypeStruct((2 * N, DHH), jnp.float32),
    mesh=_mesh,
    compiler_params=pltpu.CompilerParams(use_tc_tiling_on_sc=False),
    scratch_types=[
        pltpu.VMEM((DCJ, 128), jnp.int32),
        pltpu.VMEM((128, DHH), jnp.float32),
        pltpu.VMEM_SHARED((SP_ROWS, DHH), jnp.float32),
    ],
)(_deg_body)


def _pool_body(hid2, bounds, pooled4, bbuf, rbuf, resmax, resmean):
    c = lax.axis_index("c")
    s = lax.axis_index("s")
    pltpu.sync_copy(bounds.at[pl.ds(s * GPT, 48)], bbuf)

    lane = lax.broadcasted_iota(jnp.int32, (16,), 0)

    def extract(k):
        blk = bbuf[pl.ds((k // 16) * 16, 16)]
        sel = lane == lax.broadcast(k % 16, (16,))
        return jnp.sum(jnp.where(sel, blk, 0))

    neg = jnp.full((16,), -jnp.inf, dtype=jnp.float32)
    zero = jnp.zeros((16,), jnp.float32)

    def graph(k, carry):
        start = extract(k)
        end = extract(k + 1)
        nch = (end - start + CR - 1) // CR

        def chunk(t, acc):
            ofs = jnp.minimum(start + t * CR, N - CR)
            lo = start + t * CR
            pltpu.sync_copy(hid2.at[pl.ds(c * N + ofs, CR)], rbuf)

            def row(r, acc2):
                m0, m1, s0, s1 = acc2
                rid = ofs + r
                valid = jnp.logical_and(rid >= lo, rid < end)
                vb = lax.broadcast(valid, (16,))
                v0 = rbuf[r, pl.ds(0, 16)]
                v1 = rbuf[r, pl.ds(16, 16)]
                m0 = jnp.maximum(m0, jnp.where(vb, v0, neg))
                m1 = jnp.maximum(m1, jnp.where(vb, v1, neg))
                s0 = s0 + jnp.where(vb, v0, zero)
                s1 = s1 + jnp.where(vb, v1, zero)
                return (m0, m1, s0, s1)

            return lax.fori_loop(0, CR, row, acc)

        m0, m1, s0, s1 = lax.fori_loop(0, nch, chunk, (neg, neg, zero, zero))
        resmax[k, pl.ds(0, 16)] = m0
        resmax[k, pl.ds(16, 16)] = m1
        resmean[k, pl.ds(0, 16)] = s0
        resmean[k, pl.ds(16, 16)] = s1
        return carry

    lax.fori_loop(0, GPT, graph, 0)
    out_base = c * G + s * GPT
    pltpu.sync_copy(resmax, pooled4.at[pl.ds(out_base, GPT)])
    pltpu.sync_copy(resmean, pooled4.at[pl.ds(2 * G + out_base, GPT)])


_pool = functools.partial(
    pl.kernel,
    out_type=jax.ShapeDtypeStruct((4 * G, HH), jnp.float32),
    mesh=_mesh,
    compiler_params=pltpu.CompilerParams(use_tc_tiling_on_sc=False,
                                         needs_layout_passes=False),
    scratch_types=[
        pltpu.VMEM((48,), jnp.int32),
        pltpu.VMEM((CR, HH), jnp.float32),
        pltpu.VMEM((GPT, HH), jnp.float32),
        pltpu.VMEM((GPT, HH), jnp.float32),
    ],
)(_pool_body)


R = 1000


def _tck0_body(x_ref, dega_ref, degb_ref, wa_ref, wb_ref, g_ref, dinv_ref):
    deg = dega_ref[...] + degb_ref[...] - 1.0
    dinv = lax.rsqrt(jnp.maximum(deg, 1.0))
    xb = x_ref[...]
    g_ref[0] = dinv * jnp.dot(xb, wa_ref[...], preferred_element_type=jnp.float32)
    g_ref[1] = dinv * jnp.dot(xb, wb_ref[...], preferred_element_type=jnp.float32)
    dinv_ref[...] = dinv


def _tck0(x, dega, degb, w0a, w0b):
    return pl.pallas_call(
        _tck0_body,
        grid=(N // R,),
        in_specs=[
            pl.BlockSpec((R, DIN), lambda i: (i, 0)),
            pl.BlockSpec((R, 1), lambda i: (i, 0)),
            pl.BlockSpec((R, 1), lambda i: (i, 0)),
            pl.BlockSpec((DIN, HH), lambda i: (0, 0)),
            pl.BlockSpec((DIN, HH), lambda i: (0, 0)),
        ],
        out_specs=[
            pl.BlockSpec((2, R, HH), lambda i: (0, i, 0)),
            pl.BlockSpec((R, 1), lambda i: (i, 0)),
        ],
        out_shape=[
            jax.ShapeDtypeStruct((2, N, HH), jnp.float32),
            jax.ShapeDtypeStruct((N, 1), jnp.float32),
        ],
    )(x, dega, degb, w0a, w0b)


def _tckmid_body(a_ref, dinv_ref, ba_ref, bb_ref, waa, wab, wba, wbb, g_ref):
    dinv = dinv_ref[...]
    xa = jnp.tanh(dinv * a_ref[0] + ba_ref[...])
    xb = jnp.tanh(dinv * a_ref[1] + bb_ref[...])
    ya = (jnp.dot(xa, waa[...], preferred_element_type=jnp.float32)
          + jnp.dot(xb, wba[...], preferred_element_type=jnp.float32))
    yb = (jnp.dot(xa, wab[...], preferred_element_type=jnp.float32)
          + jnp.dot(xb, wbb[...], preferred_element_type=jnp.float32))
    g_ref[0] = dinv * ya
    g_ref[1] = dinv * yb


def _tckmid(a3, dinv, b, W):
    return pl.pallas_call(
        _tckmid_body,
        grid=(N // R,),
        in_specs=[
            pl.BlockSpec((2, R, HH), lambda i: (0, i, 0)),
            pl.BlockSpec((R, 1), lambda i: (i, 0)),
            pl.BlockSpec((1, HH), lambda i: (0, 0)),
            pl.BlockSpec((1, HH), lambda i: (0, 0)),
            pl.BlockSpec((HH, HH), lambda i: (0, 0)),
            pl.BlockSpec((HH, HH), lambda i: (0, 0)),
            pl.BlockSpec((HH, HH), lambda i: (0, 0)),
            pl.BlockSpec((HH, HH), lambda i: (0, 0)),
        ],
        out_specs=pl.BlockSpec((2, R, HH), lambda i: (0, i, 0)),
        out_shape=jax.ShapeDtypeStruct((2, N, HH), jnp.float32),
    )(a3, dinv, b[:HH].reshape(1, HH), b[HH:].reshape(1, HH),
      W[:HH, :HH], W[:HH, HH:], W[HH:, :HH], W[HH:, HH:])


def _tckf_body(a_ref, dinv_ref, ba_ref, bb_ref, h_ref):
    dinv = dinv_ref[...]
    h_ref[0] = jnp.tanh(dinv * a_ref[0] + ba_ref[...])
    h_ref[1] = jnp.tanh(dinv * a_ref[1] + bb_ref[...])


def _tckf(a3, dinv, b):
    return pl.pallas_call(
        _tckf_body,
        grid=(N // R,),
        in_specs=[
            pl.BlockSpec((2, R, HH), lambda i: (0, i, 0)),
            pl.BlockSpec((R, 1), lambda i: (i, 0)),
            pl.BlockSpec((1, HH), lambda i: (0, 0)),
            pl.BlockSpec((1, HH), lambda i: (0, 0)),
        ],
        out_specs=pl.BlockSpec((2, R, HH), lambda i: (0, i, 0)),
        out_shape=jax.ShapeDtypeStruct((2, N, HH), jnp.float32),
    )(a3, dinv, b[:HH].reshape(1, HH), b[HH:].reshape(1, HH))


def _tckt_body(p_ref, r_ref, w_ref, b_ref, pooled_ref, out_ref):
    p4 = p_ref[...]
    r = r_ref[...]
    pooled = jnp.concatenate(
        [p4[0:G], p4[G:2 * G], p4[2 * G:3 * G] * r, p4[3 * G:] * r], axis=1)
    pooled_ref[...] = pooled
    out_ref[...] = (jnp.dot(pooled, w_ref[...],
                            preferred_element_type=jnp.float32) + b_ref[...])


def _tckt(pooled4, r, Wout, bout):
    return pl.pallas_call(
        _tckt_body,
        out_shape=[
            jax.ShapeDtypeStruct((G, 2 * H), jnp.float32),
            jax.ShapeDtypeStruct((G, 1), jnp.float32),
        ],
    )(pooled4, r, Wout, bout.reshape(1, 1))


def kernel(x, edge_index, batch_index, W0, b0, W1, b1, W2, b2, W3, b3, Wout, bout):
    src = edge_index[0].astype(jnp.int32)
    dst = edge_index[1].astype(jnp.int32)
    bi = batch_index.astype(jnp.int32)

    pad = EP - E
    src_p = jnp.concatenate([src, jnp.zeros((pad,), jnp.int32)])
    src2 = jnp.concatenate([src_p, src_p + N]).reshape(2 * EP // 128, 128)
    dstp = jnp.concatenate([dst, jnp.full((pad,), N, jnp.int32)]).reshape(EP // 128, 128)
    bounds = jnp.searchsorted(bi, jnp.arange(G + 1, dtype=jnp.int32)).astype(jnp.int32)
    bounds_p = jnp.concatenate([bounds, jnp.full((15,), N, jnp.int32)])

    degw = _deg(dstp)
    g0, dinv = _tck0(x, degw[:N, :1], degw[N:, :1], W0[:, :HH], W0[:, HH:])
    a = _agg(g0.reshape(2 * N, HH), src2, dstp)
    g1 = _tckmid(a.reshape(2, N, HH), dinv, b0, W1)
    a = _agg(g1.reshape(2 * N, HH), src2, dstp)
    g2 = _tckmid(a.reshape(2, N, HH), dinv, b1, W2)
    a = _agg(g2.reshape(2 * N, HH), src2, dstp)
    g3 = _tckmid(a.reshape(2, N, HH), dinv, b2, W3)
    a = _agg(g3.reshape(2 * N, HH), src2, dstp)
    hid = _tckf(a.reshape(2, N, HH), dinv, b3)

    pooled4 = _pool(hid.reshape(2 * N, HH), bounds_p)
    cnt = (bounds[1:] - bounds[:-1]).astype(jnp.float32)
    r = (1.0 / jnp.maximum(cnt, 1.0)).reshape(G, 1)
    pooled, out = _tckt(pooled4, r, Wout, bout)
    return (out, pooled)


# speedup vs baseline: 1.1524x; 1.0015x over previous
import functools

import jax
import jax.numpy as jnp
from jax import lax
from jax.experimental import pallas as pl
from jax.experimental.pallas import tpu as pltpu
from jax.experimental.pallas import tpu_sc as plsc

N = 50000
E = 800000
DIN = 128
H = 64
G = 512
HH = H // 2

NSC = 2
NT = 16
RPT = N // NT

EP = 16 * 1024 * 49
EPT = EP // NT
ROWS_PT = EPT // 128
CJ = 4
CHUNKS = ROWS_PT // CJ

SP_ROWS = N + 8

CR = 128
GPT = G // NT

_mesh = plsc.VectorSubcoreMesh(core_axis_name="c", subcore_axis_name="s")


def _agg_body(g2, src2, dstp, out, srcvA, dstvA, srcvB, dstvB, rows, spo, sem):
    c = lax.axis_index("c")
    s = lax.axis_index("s")
    base = c * N + s * RPT
    cpi = pltpu.async_copy(g2.at[pl.ds(base, RPT)], spo.at[pl.ds(s * RPT, RPT)],
                           sem)

    src_row0 = c * (EP // 128) + s * ROWS_PT
    dst_row0 = s * ROWS_PT

    pltpu.sync_copy(src2.at[pl.ds(src_row0, CJ)], srcvA)
    pltpu.sync_copy(dstp.at[pl.ds(dst_row0, CJ)], dstvA)
    cpi.wait()
    plsc.subcore_barrier()

    def pair(k, carry):
        cps = [pltpu.async_copy(g2.at[srcvA.at[j]], rows.at[j], sem)
               for j in range(CJ)]
        pltpu.sync_copy(src2.at[pl.ds(src_row0 + (2 * k + 1) * CJ, CJ)], srcvB)
        pltpu.sync_copy(dstp.at[pl.ds(dst_row0 + (2 * k + 1) * CJ, CJ)], dstvB)
        for j in range(CJ):
            cps[j].wait()
            pltpu.sync_copy(rows.at[j], spo.at[dstvA.at[j]], add=True)
        cps = [pltpu.async_copy(g2.at[srcvB.at[j]], rows.at[j], sem)
               for j in range(CJ)]
        nxt = jnp.minimum((2 * k + 2) * CJ, (CHUNKS - 2) * CJ)
        pltpu.sync_copy(src2.at[pl.ds(src_row0 + nxt, CJ)], srcvA)
        pltpu.sync_copy(dstp.at[pl.ds(dst_row0 + nxt, CJ)], dstvA)
        for j in range(CJ):
            cps[j].wait()
            pltpu.sync_copy(rows.at[j], spo.at[dstvB.at[j]], add=True)
        return carry

    lax.fori_loop(0, CHUNKS // 2, pair, 0)
    plsc.subcore_barrier()
    pltpu.sync_copy(spo.at[pl.ds(s * RPT, RPT)], out.at[pl.ds(base, RPT)])


_agg = functools.partial(
    pl.kernel,
    out_type=jax.ShapeDtypeStruct((2 * N, HH), jnp.float32),
    mesh=_mesh,
    compiler_params=pltpu.CompilerParams(use_tc_tiling_on_sc=False),
    scratch_types=[
        pltpu.VMEM((CJ, 128), jnp.int32),
        pltpu.VMEM((CJ, 128), jnp.int32),
        pltpu.VMEM((CJ, 128), jnp.int32),
        pltpu.VMEM((CJ, 128), jnp.int32),
        pltpu.VMEM((CJ, 128, HH), jnp.float32),
        pltpu.VMEM_SHARED((SP_ROWS, HH), jnp.float32),
        pltpu.SemaphoreType.DMA,
    ],
)(_agg_body)


DHH = 16
DROWS = EP // 128 // 2
DRPT = DROWS // NT
DCJ = 4
DCHUNKS = DRPT // DCJ


def _deg_body(dstp, out, dstv, ones_rows, spo):
    c = lax.axis_index("c")
    s = lax.axis_index("s")
    one16 = jnp.ones((DHH,), jnp.float32)

    def oi(i, carry):
        ones_rows[i] = one16
        return carry

    lax.fori_loop(0, 128, oi, 0)

    def zi(i, carry):
        pltpu.sync_copy(ones_rows.at[pl.ds(0, 125)],
                        spo.at[pl.ds(s * RPT + i * 125, 125)])
        return carry

    lax.fori_loop(0, 25, zi, 0)
    plsc.subcore_barrier()

    row0 = c * DROWS + s * DRPT

    def chunk(i, carry):
        pltpu.sync_copy(dstp.at[pl.ds(row0 + i * DCJ, DCJ)], dstv)
        for j in range(DCJ):
            pltpu.sync_copy(ones_rows.at[pl.ds(0, 128)], spo.at[dstv.at[j]],
                            add=True)
        return carry

    lax.fori_loop(0, DCHUNKS, chunk, 0)
    plsc.subcore_barrier()
    pltpu.sync_copy(spo.at[pl.ds(s * RPT, RPT)], out.at[pl.ds(c * N + s * RPT, RPT)])


_deg = functools.partial(
    pl.kernel,
    out_type=jax.ShapeDtypeStruct((2 * N, DHH), jnp.float32),
    mesh=_mesh,
    compiler_params=pltpu.CompilerParams(use_tc_tiling_on_sc=False),
    scratch_types=[
        pltpu.VMEM((DCJ, 128), jnp.int32),
        pltpu.VMEM((128, DHH), jnp.float32),
        pltpu.VMEM_SHARED((SP_ROWS, DHH), jnp.float32),
    ],
)(_deg_body)


def _pool_body(hid2, bounds, pooled4, bbuf, rbuf, resmax, resmean):
    c = lax.axis_index("c")
    s = lax.axis_index("s")
    pltpu.sync_copy(bounds.at[pl.ds(s * GPT, 48)], bbuf)

    lane = lax.broadcasted_iota(jnp.int32, (16,), 0)

    def extract(k):
        blk = bbuf[pl.ds((k // 16) * 16, 16)]
        sel = lane == lax.broadcast(k % 16, (16,))
        return jnp.sum(jnp.where(sel, blk, 0))

    neg = jnp.full((16,), -jnp.inf, dtype=jnp.float32)
    zero = jnp.zeros((16,), jnp.float32)

    def graph(k, carry):
        start = extract(k)
        end = extract(k + 1)
        nch = (end - start + CR - 1) // CR

        def chunk(t, acc):
            ofs = jnp.minimum(start + t * CR, N - CR)
            lo = start + t * CR
            pltpu.sync_copy(hid2.at[pl.ds(c * N + ofs, CR)], rbuf)

            def row(r, acc2):
                m0, m1, s0, s1 = acc2
                rid = ofs + r
                valid = jnp.logical_and(rid >= lo, rid < end)
                vb = lax.broadcast(valid, (16,))
                v0 = rbuf[r, pl.ds(0, 16)]
                v1 = rbuf[r, pl.ds(16, 16)]
                m0 = jnp.maximum(m0, jnp.where(vb, v0, neg))
                m1 = jnp.maximum(m1, jnp.where(vb, v1, neg))
                s0 = s0 + jnp.where(vb, v0, zero)
                s1 = s1 + jnp.where(vb, v1, zero)
                return (m0, m1, s0, s1)

            return lax.fori_loop(0, CR, row, acc)

        m0, m1, s0, s1 = lax.fori_loop(0, nch, chunk, (neg, neg, zero, zero))
        resmax[k, pl.ds(0, 16)] = m0
        resmax[k, pl.ds(16, 16)] = m1
        resmean[k, pl.ds(0, 16)] = s0
        resmean[k, pl.ds(16, 16)] = s1
        return carry

    lax.fori_loop(0, GPT, graph, 0)
    out_base = c * G + s * GPT
    pltpu.sync_copy(resmax, pooled4.at[pl.ds(out_base, GPT)])
    pltpu.sync_copy(resmean, pooled4.at[pl.ds(2 * G + out_base, GPT)])


_pool = functools.partial(
    pl.kernel,
    out_type=jax.ShapeDtypeStruct((4 * G, HH), jnp.float32),
    mesh=_mesh,
    compiler_params=pltpu.CompilerParams(use_tc_tiling_on_sc=False,
                                         needs_layout_passes=False),
    scratch_types=[
        pltpu.VMEM((48,), jnp.int32),
        pltpu.VMEM((CR, HH), jnp.float32),
        pltpu.VMEM((GPT, HH), jnp.float32),
        pltpu.VMEM((GPT, HH), jnp.float32),
    ],
)(_pool_body)


R = 1000


def _tck0_body(x_ref, dega_ref, degb_ref, wa_ref, wb_ref, g_ref, dinv_ref):
    deg = dega_ref[...] + degb_ref[...] - 1.0
    dinv = lax.rsqrt(jnp.maximum(deg, 1.0))
    xb = x_ref[...]
    g_ref[0] = dinv * jnp.dot(xb, wa_ref[...], preferred_element_type=jnp.float32)
    g_ref[1] = dinv * jnp.dot(xb, wb_ref[...], preferred_element_type=jnp.float32)
    dinv_ref[...] = dinv


def _tck0(x, dega, degb, w0a, w0b):
    return pl.pallas_call(
        _tck0_body,
        grid=(N // R,),
        in_specs=[
            pl.BlockSpec((R, DIN), lambda i: (i, 0)),
            pl.BlockSpec((R, 1), lambda i: (i, 0)),
            pl.BlockSpec((R, 1), lambda i: (i, 0)),
            pl.BlockSpec((DIN, HH), lambda i: (0, 0)),
            pl.BlockSpec((DIN, HH), lambda i: (0, 0)),
        ],
        out_specs=[
            pl.BlockSpec((2, R, HH), lambda i: (0, i, 0)),
            pl.BlockSpec((R, 1), lambda i: (i, 0)),
        ],
        out_shape=[
            jax.ShapeDtypeStruct((2, N, HH), jnp.float32),
            jax.ShapeDtypeStruct((N, 1), jnp.float32),
        ],
    )(x, dega, degb, w0a, w0b)


def _tckmid_body(a_ref, dinv_ref, ba_ref, bb_ref, waa, wab, wba, wbb, g_ref):
    dinv = dinv_ref[...]
    xa = jnp.tanh(dinv * a_ref[0] + ba_ref[...])
    xb = jnp.tanh(dinv * a_ref[1] + bb_ref[...])
    ya = (jnp.dot(xa, waa[...], preferred_element_type=jnp.float32)
          + jnp.dot(xb, wba[...], preferred_element_type=jnp.float32))
    yb = (jnp.dot(xa, wab[...], preferred_element_type=jnp.float32)
          + jnp.dot(xb, wbb[...], preferred_element_type=jnp.float32))
    g_ref[0] = dinv * ya
    g_ref[1] = dinv * yb


def _tckmid(a3, dinv, b, W):
    return pl.pallas_call(
        _tckmid_body,
        grid=(N // R,),
        in_specs=[
            pl.BlockSpec((2, R, HH), lambda i: (0, i, 0)),
            pl.BlockSpec((R, 1), lambda i: (i, 0)),
            pl.BlockSpec((1, HH), lambda i: (0, 0)),
            pl.BlockSpec((1, HH), lambda i: (0, 0)),
            pl.BlockSpec((HH, HH), lambda i: (0, 0)),
            pl.BlockSpec((HH, HH), lambda i: (0, 0)),
            pl.BlockSpec((HH, HH), lambda i: (0, 0)),
            pl.BlockSpec((HH, HH), lambda i: (0, 0)),
        ],
        out_specs=pl.BlockSpec((2, R, HH), lambda i: (0, i, 0)),
        out_shape=jax.ShapeDtypeStruct((2, N, HH), jnp.float32),
    )(a3, dinv, b[:HH].reshape(1, HH), b[HH:].reshape(1, HH),
      W[:HH, :HH], W[:HH, HH:], W[HH:, :HH], W[HH:, HH:])


def _tckf_body(a_ref, dinv_ref, ba_ref, bb_ref, h_ref):
    dinv = dinv_ref[...]
    h_ref[0] = jnp.tanh(dinv * a_ref[0] + ba_ref[...])
    h_ref[1] = jnp.tanh(dinv * a_ref[1] + bb_ref[...])


def _tckf(a3, dinv, b):
    return pl.pallas_call(
        _tckf_body,
        grid=(N // R,),
        in_specs=[
            pl.BlockSpec((2, R, HH), lambda i: (0, i, 0)),
            pl.BlockSpec((R, 1), lambda i: (i, 0)),
            pl.BlockSpec((1, HH), lambda i: (0, 0)),
            pl.BlockSpec((1, HH), lambda i: (0, 0)),
        ],
        out_specs=pl.BlockSpec((2, R, HH), lambda i: (0, i, 0)),
        out_shape=jax.ShapeDtypeStruct((2, N, HH), jnp.float32),
    )(a3, dinv, b[:HH].reshape(1, HH), b[HH:].reshape(1, HH))


def _tckt_body(p_ref, r_ref, w_ref, b_ref, pooled_ref, out_ref):
    p4 = p_ref[...]
    r = r_ref[...]
    pooled = jnp.concatenate(
        [p4[0:G], p4[G:2 * G], p4[2 * G:3 * G] * r, p4[3 * G:] * r], axis=1)
    pooled_ref[...] = pooled
    out_ref[...] = (jnp.dot(pooled, w_ref[...],
                            preferred_element_type=jnp.float32) + b_ref[...])


def _tckt(pooled4, r, Wout, bout):
    return pl.pallas_call(
        _tckt_body,
        out_shape=[
            jax.ShapeDtypeStruct((G, 2 * H), jnp.float32),
            jax.ShapeDtypeStruct((G, 1), jnp.float32),
        ],
    )(pooled4, r, Wout, bout.reshape(1, 1))


def kernel(x, edge_index, batch_index, W0, b0, W1, b1, W2, b2, W3, b3, Wout, bout):
    src = edge_index[0].astype(jnp.int32)
    dst = edge_index[1].astype(jnp.int32)
    bi = batch_index.astype(jnp.int32)

    pad = EP - E
    src_p = jnp.concatenate([src, jnp.zeros((pad,), jnp.int32)])
    src2 = jnp.concatenate([src_p, src_p + N]).reshape(2 * EP // 128, 128)
    dstp = jnp.concatenate([dst, jnp.full((pad,), N, jnp.int32)]).reshape(EP // 128, 128)
    bounds = jnp.searchsorted(bi, jnp.arange(G + 1, dtype=jnp.int32)).astype(jnp.int32)
    bounds_p = jnp.concatenate([bounds, jnp.full((15,), N, jnp.int32)])

    degw = _deg(dstp)
    g0, dinv = _tck0(x, degw[:N, :1], degw[N:, :1], W0[:, :HH], W0[:, HH:])
    a = _agg(g0.reshape(2 * N, HH), src2, dstp)
    g1 = _tckmid(a.reshape(2, N, HH), dinv, b0, W1)
    a = _agg(g1.reshape(2 * N, HH), src2, dstp)
    g2 = _tckmid(a.reshape(2, N, HH), dinv, b1, W2)
    a = _agg(g2.reshape(2 * N, HH), src2, dstp)
    g3 = _tckmid(a.reshape(2, N, HH), dinv, b2, W3)
    a = _agg(g3.reshape(2 * N, HH), src2, dstp)
    hid = _tckf(a.reshape(2, N, HH), dinv, b3)

    pooled4 = _pool(hid.reshape(2 * N, HH), bounds_p)
    cnt = (bounds[1:] - bounds[:-1]).astype(jnp.float32)
    r = (1.0 / jnp.maximum(cnt, 1.0)).reshape(G, 1)
    pooled, out = _tckt(pooled4, r, Wout, bout)
    return (out, pooled)
